# baseline (speedup 1.0000x reference)
import functools

import jax
import jax.numpy as jnp
from jax import lax
from jax.experimental import pallas as pl
from jax.experimental.pallas import tpu as pltpu
from jax.experimental.pallas import tpu_sc as plsc

B = 8
L = 10000
E = 160000
H = 128

NC = 2
NS = 16
CHUNK = 128
CHUNKS_PER_TILE = 80
E_PAD = NS * CHUNKS_PER_TILE * CHUNK
L_PAD = 10240
ROWS_PER_TILE = 640
B_PER_CORE = B // NC

_RB = 1000
_GRID = (B * L) // _RB

_INV_SQRT2 = 0.7071067811865476


def _gelu(x):
    return 0.5 * x * (1.0 + lax.erf(x * _INV_SQRT2))


def _phase1_body(x_ref, w1_ref, b1_ref, g1_ref, bb1_ref, mw_ref, mb_ref,
                 out_ref):
    x = x_ref[...]
    h = x * w1_ref[...] + b1_ref[...]
    mu = jnp.mean(h, axis=-1, keepdims=True)
    var = jnp.mean((h - mu) ** 2, axis=-1, keepdims=True)
    h = (h - mu) * lax.rsqrt(var + 1e-5) * g1_ref[...] + bb1_ref[...]
    h = _gelu(h)
    out_ref[...] = (
        jnp.dot(h, mw_ref[...], preferred_element_type=jnp.float32)
        + mb_ref[...]
    ).astype(jnp.bfloat16)


def _phase1(x2d, fc1_w, fc1_b, ln1_g, ln1_b, msg_w, msg_b):
    full = lambda i: (0, 0)
    row = lambda i: (i, 0)
    return pl.pallas_call(
        _phase1_body,
        grid=(_GRID,),
        in_specs=[
            pl.BlockSpec((_RB, 1), row),
            pl.BlockSpec((1, H), full),
            pl.BlockSpec((1, H), full),
            pl.BlockSpec((1, H), full),
            pl.BlockSpec((1, H), full),
            pl.BlockSpec((H, H), full),
            pl.BlockSpec((1, H), full),
        ],
        out_specs=pl.BlockSpec((_RB, H), row),
        out_shape=jax.ShapeDtypeStruct((B * L, H), jnp.bfloat16),
    )(x2d, fc1_w, fc1_b, ln1_g, ln1_b, msg_w, msg_b)


HH = H // 2


NSLOT = 5


def _sc_body(msg_hbm, src_hbm, dst_hbm, zerosh_hbm, zeros16_hbm, ones16_hbm,
             agg_hbm, deg_hbm,
             src_v, dst_v, gbuf_v, zerosh_v, zeros16_v, ones16_v,
             gsem_a, ssem_a, agg_sh, deg_sh):
    gsems = [gsem_a.at[t] for t in range(NSLOT)]
    ssems = [ssem_a.at[t] for t in range(NSLOT)]
    c = lax.axis_index("c")
    s = lax.axis_index("s")
    row0 = s * ROWS_PER_TILE
    n_last = L - (NS - 1) * ROWS_PER_TILE

    pltpu.sync_copy(dst_hbm.at[s], dst_v)
    pltpu.sync_copy(zerosh_hbm, zerosh_v)
    pltpu.sync_copy(zeros16_hbm, zeros16_v)
    pltpu.sync_copy(ones16_hbm, ones16_v)

    @pl.when(c == 0)
    def _deg():
        for j in range(5):
            pltpu.sync_copy(zeros16_v, deg_sh.at[pl.ds(row0 + j * CHUNK, CHUNK)])
        plsc.subcore_barrier()

        def deg_chunk(i, carry):
            pltpu.sync_copy(ones16_v, deg_sh.at[dst_v.at[i]], add=True)
            return carry
        lax.fori_loop(0, CHUNKS_PER_TILE, deg_chunk, 0)
        plsc.subcore_barrier()

        @pl.when(s < NS - 1)
        def _():
            pltpu.sync_copy(deg_sh.at[pl.ds(row0, ROWS_PER_TILE)],
                            deg_hbm.at[pl.ds(row0, ROWS_PER_TILE)])

        @pl.when(s == NS - 1)
        def _():
            pltpu.sync_copy(deg_sh.at[pl.ds(row0, n_last)],
                            deg_hbm.at[pl.ds(row0, n_last)])

    for bb in range(B_PER_CORE):
        if True:
            b = c * B_PER_CORE + bb
            out_hbm = agg_hbm

            for j in range(5):
                pltpu.sync_copy(zerosh_v,
                                agg_sh.at[pl.ds(row0 + j * CHUNK, CHUNK)])
            pltpu.sync_copy(src_hbm.at[b * NS + s], src_v)
            plsc.subcore_barrier()

            def _slot(t):
                return gbuf_v.at[pl.ds(t * CHUNK, CHUNK)]

            def _gissue(j, t):
                pltpu.async_copy(msg_hbm.at[src_v.at[j]], _slot(t), gsems[t])

            def _gwait(j, t):
                pltpu.make_async_copy(msg_hbm.at[src_v.at[j]], _slot(t),
                                      gsems[t]).wait()

            def _sissue(j, t):
                pltpu.async_copy(_slot(t), agg_sh.at[dst_v.at[j]], ssems[t],
                                 add=True)

            def _swait(j, t):
                pltpu.make_async_copy(_slot(t), agg_sh.at[dst_v.at[j]],
                                      ssems[t]).wait()

            for t in range(NSLOT):
                _gissue(t, t)

            def edge_step(i, carry):
                base = i * NSLOT
                for t in range(NSLOT):
                    _gwait(base + t, t)
                    _sissue(base + t, t)

                @pl.when(i < CHUNKS_PER_TILE // NSLOT - 1)
                def _():
                    for t in range(NSLOT):
                        _swait(base + t, t)
                        _gissue(base + NSLOT + t, t)
                return carry
            lax.fori_loop(0, CHUNKS_PER_TILE // NSLOT, edge_step, 0)
            for t in range(NSLOT):
                _swait(CHUNKS_PER_TILE - NSLOT + t, t)
            plsc.subcore_barrier()

            hbase = b * L + row0

            @pl.when(s < NS - 1)
            def _():
                pltpu.sync_copy(agg_sh.at[pl.ds(row0, ROWS_PER_TILE)],
                                out_hbm.at[pl.ds(hbase, ROWS_PER_TILE)])

            @pl.when(s == NS - 1)
            def _():
                pltpu.sync_copy(agg_sh.at[pl.ds(row0, n_last)],
                                out_hbm.at[pl.ds(hbase, n_last)])

            plsc.subcore_barrier()


def _phase2(msgbf, src_abs, dst_tiles, zerosh, zeros16, ones16):
    mesh = plsc.VectorSubcoreMesh(core_axis_name="c", subcore_axis_name="s")
    f = pl.kernel(
        _sc_body,
        out_type=[
            jax.ShapeDtypeStruct((B * L, H), jnp.bfloat16),
            jax.ShapeDtypeStruct((L, 16), jnp.float32),
        ],
        mesh=mesh,
        scratch_types=[
            pltpu.VMEM((CHUNKS_PER_TILE, CHUNK), jnp.int32),
            pltpu.VMEM((CHUNKS_PER_TILE, CHUNK), jnp.int32),
            pltpu.VMEM((NSLOT * CHUNK, H), jnp.bfloat16),
            pltpu.VMEM((CHUNK, H), jnp.bfloat16),
            pltpu.VMEM((CHUNK, 16), jnp.float32),
            pltpu.VMEM((CHUNK, 16), jnp.float32),
            pltpu.SemaphoreType.DMA((NSLOT,)),
            pltpu.SemaphoreType.DMA((NSLOT,)),
            pltpu.VMEM_SHARED((L_PAD, H), jnp.bfloat16),
            pltpu.VMEM_SHARED((L_PAD, 16), jnp.float32),
        ],
        compiler_params=pltpu.CompilerParams(use_tc_tiling_on_sc=False),
    )
    return f(msgbf, src_abs, dst_tiles, zerosh, zeros16, ones16)


def _phase3_body(x_ref, agg_ref, deg_ref,
                 w1_ref, b1_ref, g1_ref, bb1_ref,
                 uwh_ref, uwa_ref, ub_ref,
                 f2w_ref, f2b_ref, g2_ref, bb2_ref,
                 ow_ref, ob_ref, sk_ref, out_ref):
    x = x_ref[...]
    h = x * w1_ref[...] + b1_ref[...]
    mu = jnp.mean(h, axis=-1, keepdims=True)
    var = jnp.mean((h - mu) ** 2, axis=-1, keepdims=True)
    h = (h - mu) * lax.rsqrt(var + 1e-5) * g1_ref[...] + bb1_ref[...]
    h = _gelu(h)

    a = agg_ref[...].astype(jnp.float32) / jnp.maximum(deg_ref[...], 1.0)
    u = (jnp.dot(h, uwh_ref[...], preferred_element_type=jnp.float32)
         + jnp.dot(a, uwa_ref[...], preferred_element_type=jnp.float32)
         + ub_ref[...])
    u = _gelu(u)

    h2 = jnp.dot(u, f2w_ref[...], preferred_element_type=jnp.float32) + f2b_ref[...]
    mu2 = jnp.mean(h2, axis=-1, keepdims=True)
    var2 = jnp.mean((h2 - mu2) ** 2, axis=-1, keepdims=True)
    h2 = (h2 - mu2) * lax.rsqrt(var2 + 1e-5) * g2_ref[...] + bb2_ref[...]
    h2 = h2 + u
    h2 = _gelu(h2)

    refined = jnp.sum(h2 * ow_ref[...], axis=-1, keepdims=True) + ob_ref[...]
    alpha = jax.nn.sigmoid(sk_ref[...])
    out_ref[...] = alpha * refined + (1.0 - alpha) * x


def _phase3(x2d, agg, deg2d, fc1_w, fc1_b, ln1_g, ln1_b,
            upd_wh, upd_wa, upd_b, fc2_w, fc2_b, ln2_g, ln2_b,
            out_w_row, out_b, skip_w2d):
    full = lambda i: (0, 0)
    row = lambda i: (i, 0)
    return pl.pallas_call(
        _phase3_body,
        grid=(_GRID,),
        in_specs=[
            pl.BlockSpec((_RB, 1), row),
            pl.BlockSpec((_RB, H), row),
            pl.BlockSpec((_RB, 1), lambda i: (i % (L // _RB), 0)),
            pl.BlockSpec((1, H), full),
            pl.BlockSpec((1, H), full),
            pl.BlockSpec((1, H), full),
            pl.BlockSpec((1, H), full),
            pl.BlockSpec((H, H), full),
            pl.BlockSpec((H, H), full),
            pl.BlockSpec((1, H), full),
            pl.BlockSpec((H, H), full),
            pl.BlockSpec((1, H), full),
            pl.BlockSpec((1, H), full),
            pl.BlockSpec((1, H), full),
            pl.BlockSpec((1, H), full),
            pl.BlockSpec((1, 1), full),
            pl.BlockSpec((1, 1), full),
        ],
        out_specs=pl.BlockSpec((_RB, 1), row),
        out_shape=jax.ShapeDtypeStruct((B * L, 1), jnp.float32),
    )(x2d, agg, deg2d, fc1_w, fc1_b, ln1_g, ln1_b,
      upd_wh, upd_wa, upd_b, fc2_w, fc2_b, ln2_g, ln2_b,
      out_w_row, out_b, skip_w2d)


def kernel(logits, edge_index, fc1_w, fc1_b, ln1_g, ln1_b, msg_w, msg_b,
           upd_w, upd_b, fc2_w, fc2_b, ln2_g, ln2_b, out_w, out_b, skip_w):
    x2d = logits.reshape(B * L, 1)
    r = lambda v: v.reshape(1, H)

    msg_flat = _phase1(x2d, fc1_w.reshape(1, H), r(fc1_b), r(ln1_g), r(ln1_b),
                       msg_w, r(msg_b))

    src = edge_index[0]
    dst = edge_index[1]
    pad = E_PAD - E
    src_p = jnp.concatenate([src, jnp.zeros((pad,), jnp.int32)])
    dst_p = jnp.concatenate([dst, jnp.full((pad,), L, jnp.int32)])
    src_tiles = src_p.reshape(NS, CHUNKS_PER_TILE, CHUNK)
    src_abs = (src_tiles[None] +
               (jnp.arange(B, dtype=jnp.int32) * L)[:, None, None, None])
    src_abs = src_abs.reshape(B * NS, CHUNKS_PER_TILE, CHUNK)
    dst_tiles = dst_p.reshape(NS, CHUNKS_PER_TILE, CHUNK)

    zerosh = jnp.zeros((CHUNK, H), jnp.bfloat16)
    zeros16 = jnp.zeros((CHUNK, 16), jnp.float32)
    ones16 = jnp.ones((CHUNK, 16), jnp.float32)

    agg, deg16 = _phase2(msg_flat, src_abs, dst_tiles,
                         zerosh, zeros16, ones16)
    deg2d = deg16[:, :1]

    refined = _phase3(
        x2d, agg, deg2d,
        fc1_w.reshape(1, H), r(fc1_b), r(ln1_g), r(ln1_b),
        upd_w[:H], upd_w[H:], r(upd_b),
        fc2_w, r(fc2_b), r(ln2_g), r(ln2_b),
        out_w.reshape(1, H), out_b.reshape(1, 1), skip_w.reshape(1, 1),
    )
    return refined.reshape(B, L)


# speedup vs baseline: 1.0195x; 1.0195x over previous
import functools

import jax
import jax.numpy as jnp
from jax import lax
from jax.experimental import pallas as pl
from jax.experimental.pallas import tpu as pltpu
from jax.experimental.pallas import tpu_sc as plsc

B = 8
L = 10000
E = 160000
H = 128

NC = 2
NS = 16
CHUNK = 128
CHUNKS_PER_TILE = 80
E_PAD = NS * CHUNKS_PER_TILE * CHUNK
L_PAD = 10240
ROWS_PER_TILE = 640
B_PER_CORE = B // NC

_RB = 1000
_GRID = (B * L) // _RB

_INV_SQRT2 = 0.7071067811865476


def _gelu(x):
    return 0.5 * x * (1.0 + lax.erf(x * _INV_SQRT2))


def _node_head(x, wcg_ref, bcg_ref, bb1_ref, s3_ref):
    sww = s3_ref[0, 0]
    swb = s3_ref[0, 1]
    sbb = s3_ref[0, 2]
    r = lax.rsqrt((x * x) * sww + (2.0 * x) * swb + (sbb + 1e-5))
    h = (x * r) * wcg_ref[...] + r * bcg_ref[...] + bb1_ref[...]
    return _gelu(h)


def _phase1_body(x_ref, wcg_ref, bcg_ref, bb1_ref, s3_ref, out_ref):
    out_ref[...] = _node_head(x_ref[...], wcg_ref, bcg_ref, bb1_ref,
                              s3_ref).astype(jnp.bfloat16)


def _phase1(x2d, wcg, bcg, ln1_b, s3):
    full = lambda i: (0, 0)
    row = lambda i: (i, 0)
    return pl.pallas_call(
        _phase1_body,
        grid=(_GRID,),
        in_specs=[
            pl.BlockSpec((_RB, 1), row),
            pl.BlockSpec((1, H), full),
            pl.BlockSpec((1, H), full),
            pl.BlockSpec((1, H), full),
            pl.BlockSpec((1, 3), full),
        ],
        out_specs=pl.BlockSpec((_RB, H), row),
        out_shape=jax.ShapeDtypeStruct((B * L, H), jnp.bfloat16),
    )(x2d, wcg, bcg, ln1_b, s3)


HH = H // 2


NSLOT = 5


def _sc_body(msg_hbm, src_hbm, dst_hbm, zerosh_hbm, zeros16_hbm, ones16_hbm,
             agg_hbm, deg_hbm,
             src_v, dst_v, gbuf_v, zerosh_v, zeros16_v, ones16_v,
             gsem_a, ssem_a, agg_sh, deg_sh):
    gsems = [gsem_a.at[t] for t in range(NSLOT)]
    ssems = [ssem_a.at[t] for t in range(NSLOT)]
    c = lax.axis_index("c")
    s = lax.axis_index("s")
    row0 = s * ROWS_PER_TILE
    n_last = L - (NS - 1) * ROWS_PER_TILE

    pltpu.sync_copy(dst_hbm.at[s], dst_v)
    pltpu.sync_copy(zerosh_hbm, zerosh_v)
    pltpu.sync_copy(zeros16_hbm, zeros16_v)
    pltpu.sync_copy(ones16_hbm, ones16_v)

    @pl.when(c == 0)
    def _deg():
        for j in range(5):
            pltpu.sync_copy(zeros16_v, deg_sh.at[pl.ds(row0 + j * CHUNK, CHUNK)])
        plsc.subcore_barrier()

        def deg_chunk(i, carry):
            pltpu.sync_copy(ones16_v, deg_sh.at[dst_v.at[i]], add=True)
            return carry
        lax.fori_loop(0, CHUNKS_PER_TILE, deg_chunk, 0)
        plsc.subcore_barrier()

        @pl.when(s < NS - 1)
        def _():
            pltpu.sync_copy(deg_sh.at[pl.ds(row0, ROWS_PER_TILE)],
                            deg_hbm.at[pl.ds(row0, ROWS_PER_TILE)])

        @pl.when(s == NS - 1)
        def _():
            pltpu.sync_copy(deg_sh.at[pl.ds(row0, n_last)],
                            deg_hbm.at[pl.ds(row0, n_last)])

    for bb in range(B_PER_CORE):
        if True:
            b = c * B_PER_CORE + bb
            out_hbm = agg_hbm

            for j in range(5):
                pltpu.sync_copy(zerosh_v,
                                agg_sh.at[pl.ds(row0 + j * CHUNK, CHUNK)])
            pltpu.sync_copy(src_hbm.at[b * NS + s], src_v)
            plsc.subcore_barrier()

            def _slot(t):
                return gbuf_v.at[pl.ds(t * CHUNK, CHUNK)]

            def _gissue(j, t):
                pltpu.async_copy(msg_hbm.at[src_v.at[j]], _slot(t), gsems[t])

            def _gwait(j, t):
                pltpu.make_async_copy(msg_hbm.at[src_v.at[j]], _slot(t),
                                      gsems[t]).wait()

            def _sissue(j, t):
                pltpu.async_copy(_slot(t), agg_sh.at[dst_v.at[j]], ssems[t],
                                 add=True)

            def _swait(j, t):
                pltpu.make_async_copy(_slot(t), agg_sh.at[dst_v.at[j]],
                                      ssems[t]).wait()

            for t in range(NSLOT):
                _gissue(t, t)

            def edge_step(i, carry):
                base = i * NSLOT
                for t in range(NSLOT):
                    _gwait(base + t, t)
                    _sissue(base + t, t)

                @pl.when(i < CHUNKS_PER_TILE // NSLOT - 1)
                def _():
                    for t in range(NSLOT):
                        _swait(base + t, t)
                        _gissue(base + NSLOT + t, t)
                return carry
            lax.fori_loop(0, CHUNKS_PER_TILE // NSLOT, edge_step, 0)
            for t in range(NSLOT):
                _swait(CHUNKS_PER_TILE - NSLOT + t, t)
            plsc.subcore_barrier()

            hbase = b * L + row0

            @pl.when(s < NS - 1)
            def _():
                pltpu.sync_copy(agg_sh.at[pl.ds(row0, ROWS_PER_TILE)],
                                out_hbm.at[pl.ds(hbase, ROWS_PER_TILE)])

            @pl.when(s == NS - 1)
            def _():
                pltpu.sync_copy(agg_sh.at[pl.ds(row0, n_last)],
                                out_hbm.at[pl.ds(hbase, n_last)])

            plsc.subcore_barrier()


def _phase2(msgbf, src_abs, dst_tiles, zerosh, zeros16, ones16):
    mesh = plsc.VectorSubcoreMesh(core_axis_name="c", subcore_axis_name="s")
    f = pl.kernel(
        _sc_body,
        out_type=[
            jax.ShapeDtypeStruct((B * L, H), jnp.bfloat16),
            jax.ShapeDtypeStruct((L, 16), jnp.float32),
        ],
        mesh=mesh,
        scratch_types=[
            pltpu.VMEM((CHUNKS_PER_TILE, CHUNK), jnp.int32),
            pltpu.VMEM((CHUNKS_PER_TILE, CHUNK), jnp.int32),
            pltpu.VMEM((NSLOT * CHUNK, H), jnp.bfloat16),
            pltpu.VMEM((CHUNK, H), jnp.bfloat16),
            pltpu.VMEM((CHUNK, 16), jnp.float32),
            pltpu.VMEM((CHUNK, 16), jnp.float32),
            pltpu.SemaphoreType.DMA((NSLOT,)),
            pltpu.SemaphoreType.DMA((NSLOT,)),
            pltpu.VMEM_SHARED((L_PAD, H), jnp.bfloat16),
            pltpu.VMEM_SHARED((L_PAD, 16), jnp.float32),
        ],
        compiler_params=pltpu.CompilerParams(use_tc_tiling_on_sc=False),
    )
    return f(msgbf, src_abs, dst_tiles, zerosh, zeros16, ones16)


def _phase3_body(x_ref, agg_ref, deg_ref,
                 wcg_ref, bcg_ref, bb1_ref, s3_ref,
                 uwh_ref, uwa_ref, ub_ref,
                 f2w_ref, f2b_ref, g2_ref, bb2_ref,
                 ow_ref, ob_ref, sk_ref, out_ref):
    x = x_ref[...]
    h = _node_head(x, wcg_ref, bcg_ref, bb1_ref, s3_ref)

    a = (agg_ref[...].astype(jnp.float32)
         / jnp.maximum(deg_ref[...], 1.0)).astype(jnp.bfloat16)
    u = (jnp.dot(h.astype(jnp.bfloat16), uwh_ref[...],
                 preferred_element_type=jnp.float32)
         + jnp.dot(a, uwa_ref[...], preferred_element_type=jnp.float32)
         + ub_ref[...])
    u = _gelu(u)

    h2 = jnp.dot(u.astype(jnp.bfloat16), f2w_ref[...],
                 preferred_element_type=jnp.float32) + f2b_ref[...]
    mu2 = jnp.mean(h2, axis=-1, keepdims=True)
    var2 = jnp.mean((h2 - mu2) ** 2, axis=-1, keepdims=True)
    h2 = (h2 - mu2) * lax.rsqrt(var2 + 1e-5) * g2_ref[...] + bb2_ref[...]
    h2 = h2 + u
    h2 = _gelu(h2)

    refined = jnp.sum(h2 * ow_ref[...], axis=-1, keepdims=True) + ob_ref[...]
    alpha = jax.nn.sigmoid(sk_ref[...])
    out_ref[...] = alpha * refined + (1.0 - alpha) * x


def _phase3(x2d, agg, deg2d, wcg, bcg, ln1_b, s3,
            upd_wh, upd_wa, upd_b, fc2_w, fc2_b, ln2_g, ln2_b,
            out_w_row, out_b, skip_w2d):
    full = lambda i: (0, 0)
    row = lambda i: (i, 0)
    return pl.pallas_call(
        _phase3_body,
        grid=(_GRID,),
        in_specs=[
            pl.BlockSpec((_RB, 1), row),
            pl.BlockSpec((_RB, H), row),
            pl.BlockSpec((_RB, 1), lambda i: (i % (L // _RB), 0)),
            pl.BlockSpec((1, H), full),
            pl.BlockSpec((1, H), full),
            pl.BlockSpec((1, H), full),
            pl.BlockSpec((1, 3), full),
            pl.BlockSpec((H, H), full),
            pl.BlockSpec((H, H), full),
            pl.BlockSpec((1, H), full),
            pl.BlockSpec((H, H), full),
            pl.BlockSpec((1, H), full),
            pl.BlockSpec((1, H), full),
            pl.BlockSpec((1, H), full),
            pl.BlockSpec((1, H), full),
            pl.BlockSpec((1, 1), full),
            pl.BlockSpec((1, 1), full),
        ],
        out_specs=pl.BlockSpec((_RB, 1), row),
        out_shape=jax.ShapeDtypeStruct((B * L, 1), jnp.float32),
    )(x2d, agg, deg2d, wcg, bcg, ln1_b, s3,
      upd_wh, upd_wa, upd_b, fc2_w, fc2_b, ln2_g, ln2_b,
      out_w_row, out_b, skip_w2d)


def kernel(logits, edge_index, fc1_w, fc1_b, ln1_g, ln1_b, msg_w, msg_b,
           upd_w, upd_b, fc2_w, fc2_b, ln2_g, ln2_b, out_w, out_b, skip_w):
    x2d = logits.reshape(B * L, 1)
    r = lambda v: v.reshape(1, H)

    w0 = fc1_w.reshape(H)
    wz = w0 - jnp.mean(w0)
    bz = fc1_b - jnp.mean(fc1_b)
    s3 = jnp.stack([jnp.mean(wz * wz), jnp.mean(wz * bz),
                    jnp.mean(bz * bz)]).reshape(1, 3)
    wcg = (wz * ln1_g).reshape(1, H)
    bcg = (bz * ln1_g).reshape(1, H)

    h_flat = _phase1(x2d, wcg, bcg, r(ln1_b), s3)

    src = edge_index[0]
    dst = edge_index[1]
    pad = E_PAD - E
    src_p = jnp.concatenate([src, jnp.zeros((pad,), jnp.int32)])
    dst_p = jnp.concatenate([dst, jnp.full((pad,), L, jnp.int32)])
    src_tiles = src_p.reshape(NS, CHUNKS_PER_TILE, CHUNK)
    src_abs = (src_tiles[None] +
               (jnp.arange(B, dtype=jnp.int32) * L)[:, None, None, None])
    src_abs = src_abs.reshape(B * NS, CHUNKS_PER_TILE, CHUNK)
    dst_tiles = dst_p.reshape(NS, CHUNKS_PER_TILE, CHUNK)

    zerosh = jnp.zeros((CHUNK, H), jnp.bfloat16)
    zeros16 = jnp.zeros((CHUNK, 16), jnp.float32)
    ones16 = jnp.ones((CHUNK, 16), jnp.float32)

    agg, deg16 = _phase2(h_flat, src_abs, dst_tiles,
                         zerosh, zeros16, ones16)
    deg2d = deg16[:, :1]

    uwh = upd_w[:H].astype(jnp.bfloat16)
    uwa = (msg_w @ upd_w[H:]).astype(jnp.bfloat16)
    ub = (msg_b @ upd_w[H:] + upd_b).reshape(1, H)

    refined = _phase3(
        x2d, agg, deg2d,
        wcg, bcg, r(ln1_b), s3,
        uwh, uwa, ub,
        fc2_w.astype(jnp.bfloat16), r(fc2_b), r(ln2_g), r(ln2_b),
        out_w.reshape(1, H), out_b.reshape(1, 1), skip_w.reshape(1, 1),
    )
    return refined.reshape(B, L)


# speedup vs baseline: 1.0675x; 1.0471x over previous
import functools

import jax
import jax.numpy as jnp
from jax import lax
from jax.experimental import pallas as pl
from jax.experimental.pallas import tpu as pltpu
from jax.experimental.pallas import tpu_sc as plsc

B = 8
L = 10000
E = 160000
H = 128

NC = 2
NS = 16
CHUNK = 128
CHUNKS_PER_TILE = 80
E_PAD = NS * CHUNKS_PER_TILE * CHUNK
L_PAD = 10240
ROWS_PER_TILE = 640
B_PER_CORE = B // NC

_RB = 2000
_GRID = (B * L) // _RB

_INV_SQRT2 = 0.7071067811865476


def _gelu(x):
    return 0.5 * x * (1.0 + lax.erf(x * _INV_SQRT2))


def _node_head(x, wcg_ref, bcg_ref, bb1_ref, s3_ref):
    sww = s3_ref[0, 0]
    swb = s3_ref[0, 1]
    sbb = s3_ref[0, 2]
    r = lax.rsqrt((x * x) * sww + (2.0 * x) * swb + (sbb + 1e-5))
    h = (x * r) * wcg_ref[...] + r * bcg_ref[...] + bb1_ref[...]
    return _gelu(h)


def _phase1_body(x_ref, wcg_ref, bcg_ref, bb1_ref, s3_ref, out_ref):
    out_ref[...] = _node_head(x_ref[...], wcg_ref, bcg_ref, bb1_ref,
                              s3_ref).astype(jnp.bfloat16)


def _phase1(x2d, wcg, bcg, ln1_b, s3):
    full = lambda i: (0, 0)
    row = lambda i: (i, 0)
    return pl.pallas_call(
        _phase1_body,
        grid=(_GRID,),
        in_specs=[
            pl.BlockSpec((_RB, 1), row),
            pl.BlockSpec((1, H), full),
            pl.BlockSpec((1, H), full),
            pl.BlockSpec((1, H), full),
            pl.BlockSpec((1, 3), full),
        ],
        out_specs=pl.BlockSpec((_RB, H), row),
        out_shape=jax.ShapeDtypeStruct((B * L, H), jnp.bfloat16),
    )(x2d, wcg, bcg, ln1_b, s3)


HH = H // 2


NSLOT = 4


def _sc_body(msg_hbm, src_hbm, dst_hbm, zerosh_hbm, zeros16_hbm, ones16_hbm,
             agg_hbm, deg_hbm,
             src_v, sabs_v, dst_v, gbuf_v, zerosh_v, zeros16_v, ones16_v,
             gsem_a, ssem_a, agg_sh, deg_sh):
    gsems = [gsem_a.at[t] for t in range(NSLOT)]
    ssems = [ssem_a.at[t] for t in range(NSLOT)]
    c = lax.axis_index("c")
    s = lax.axis_index("s")
    row0 = s * ROWS_PER_TILE
    n_last = L - (NS - 1) * ROWS_PER_TILE

    pltpu.sync_copy(src_hbm.at[s], src_v)
    pltpu.sync_copy(dst_hbm.at[s], dst_v)
    pltpu.sync_copy(zerosh_hbm, zerosh_v)
    pltpu.sync_copy(zeros16_hbm, zeros16_v)
    pltpu.sync_copy(ones16_hbm, ones16_v)

    @pl.when(c == 0)
    def _deg():
        for j in range(5):
            pltpu.sync_copy(zeros16_v, deg_sh.at[pl.ds(row0 + j * CHUNK, CHUNK)])
        plsc.subcore_barrier()

        def deg_chunk(i, carry):
            pltpu.sync_copy(ones16_v, deg_sh.at[dst_v.at[i]], add=True)
            return carry
        lax.fori_loop(0, CHUNKS_PER_TILE, deg_chunk, 0)
        plsc.subcore_barrier()

        @pl.when(s < NS - 1)
        def _():
            pltpu.sync_copy(deg_sh.at[pl.ds(row0, ROWS_PER_TILE)],
                            deg_hbm.at[pl.ds(row0, ROWS_PER_TILE)])

        @pl.when(s == NS - 1)
        def _():
            pltpu.sync_copy(deg_sh.at[pl.ds(row0, n_last)],
                            deg_hbm.at[pl.ds(row0, n_last)])

    for bb in range(B_PER_CORE):
        if True:
            b = c * B_PER_CORE + bb
            out_hbm = agg_hbm

            for j in range(5):
                pltpu.sync_copy(zerosh_v,
                                agg_sh.at[pl.ds(row0 + j * CHUNK, CHUNK)])

            boff = (b * L).astype(jnp.int32)

            def absrow(i, carry):
                for k in range(CHUNK // 16):
                    sabs_v[i, pl.ds(k * 16, 16)] = (
                        src_v[i, pl.ds(k * 16, 16)] + boff)
                return carry
            lax.fori_loop(0, CHUNKS_PER_TILE, absrow, 0)
            plsc.subcore_barrier()

            def _slot(t):
                return gbuf_v.at[pl.ds(t * CHUNK, CHUNK)]

            def _gissue(j, t):
                pltpu.async_copy(msg_hbm.at[sabs_v.at[j]], _slot(t), gsems[t])

            def _gwait(j, t):
                pltpu.make_async_copy(msg_hbm.at[sabs_v.at[j]], _slot(t),
                                      gsems[t]).wait()

            def _sissue(j, t):
                pltpu.async_copy(_slot(t), agg_sh.at[dst_v.at[j]], ssems[t],
                                 add=True)

            def _swait(j, t):
                pltpu.make_async_copy(_slot(t), agg_sh.at[dst_v.at[j]],
                                      ssems[t]).wait()

            for t in range(NSLOT):
                _gissue(t, t)

            def edge_step(i, carry):
                base = i * NSLOT
                for t in range(NSLOT):
                    _gwait(base + t, t)
                    _sissue(base + t, t)

                @pl.when(i < CHUNKS_PER_TILE // NSLOT - 1)
                def _():
                    for t in range(NSLOT):
                        _swait(base + t, t)
                        _gissue(base + NSLOT + t, t)
                return carry
            lax.fori_loop(0, CHUNKS_PER_TILE // NSLOT, edge_step, 0)
            for t in range(NSLOT):
                _swait(CHUNKS_PER_TILE - NSLOT + t, t)
            plsc.subcore_barrier()

            hbase = b * L + row0

            @pl.when(s < NS - 1)
            def _():
                pltpu.sync_copy(agg_sh.at[pl.ds(row0, ROWS_PER_TILE)],
                                out_hbm.at[pl.ds(hbase, ROWS_PER_TILE)])

            @pl.when(s == NS - 1)
            def _():
                pltpu.sync_copy(agg_sh.at[pl.ds(row0, n_last)],
                                out_hbm.at[pl.ds(hbase, n_last)])

            plsc.subcore_barrier()


def _phase2(msgbf, src_abs, dst_tiles, zerosh, zeros16, ones16):
    mesh = plsc.VectorSubcoreMesh(core_axis_name="c", subcore_axis_name="s")
    f = pl.kernel(
        _sc_body,
        out_type=[
            jax.ShapeDtypeStruct((B * L, H), jnp.bfloat16),
            jax.ShapeDtypeStruct((L, 16), jnp.float32),
        ],
        mesh=mesh,
        scratch_types=[
            pltpu.VMEM((CHUNKS_PER_TILE, CHUNK), jnp.int32),
            pltpu.VMEM((CHUNKS_PER_TILE, CHUNK), jnp.int32),
            pltpu.VMEM((CHUNKS_PER_TILE, CHUNK), jnp.int32),
            pltpu.VMEM((NSLOT * CHUNK, H), jnp.bfloat16),
            pltpu.VMEM((CHUNK, H), jnp.bfloat16),
            pltpu.VMEM((CHUNK, 16), jnp.float32),
            pltpu.VMEM((CHUNK, 16), jnp.float32),
            pltpu.SemaphoreType.DMA((NSLOT,)),
            pltpu.SemaphoreType.DMA((NSLOT,)),
            pltpu.VMEM_SHARED((L_PAD, H), jnp.bfloat16),
            pltpu.VMEM_SHARED((L_PAD, 16), jnp.float32),
        ],
        compiler_params=pltpu.CompilerParams(use_tc_tiling_on_sc=False),
    )
    return f(msgbf, src_abs, dst_tiles, zerosh, zeros16, ones16)


def _phase3_body(x_ref, agg_ref, deg_ref,
                 wcg_ref, bcg_ref, bb1_ref, s3_ref,
                 uwh_ref, uwa_ref, ub_ref,
                 f2w_ref, f2b_ref, g2_ref, bb2_ref,
                 ow_ref, ob_ref, sk_ref, out_ref):
    x = x_ref[...]
    h = _node_head(x, wcg_ref, bcg_ref, bb1_ref, s3_ref)

    invdeg = 1.0 / jnp.maximum(deg_ref[...], 1.0)
    m = jnp.dot(agg_ref[...], uwa_ref[...], preferred_element_type=jnp.float32)
    u = (jnp.dot(h.astype(jnp.bfloat16), uwh_ref[...],
                 preferred_element_type=jnp.float32)
         + m * invdeg + ub_ref[...])
    u = _gelu(u)

    h2 = jnp.dot(u.astype(jnp.bfloat16), f2w_ref[...],
                 preferred_element_type=jnp.float32) + f2b_ref[...]
    mu2 = jnp.mean(h2, axis=-1, keepdims=True)
    var2 = jnp.mean((h2 - mu2) ** 2, axis=-1, keepdims=True)
    h2 = (h2 - mu2) * lax.rsqrt(var2 + 1e-5) * g2_ref[...] + bb2_ref[...]
    h2 = h2 + u
    h2 = _gelu(h2)

    refined = jnp.sum(h2 * ow_ref[...], axis=-1, keepdims=True) + ob_ref[...]
    alpha = jax.nn.sigmoid(sk_ref[...])
    out_ref[...] = alpha * refined + (1.0 - alpha) * x


def _phase3(x2d, agg, deg2d, wcg, bcg, ln1_b, s3,
            upd_wh, upd_wa, upd_b, fc2_w, fc2_b, ln2_g, ln2_b,
            out_w_row, out_b, skip_w2d):
    full = lambda i: (0, 0)
    row = lambda i: (i, 0)
    return pl.pallas_call(
        _phase3_body,
        grid=(_GRID,),
        in_specs=[
            pl.BlockSpec((_RB, 1), row),
            pl.BlockSpec((_RB, H), row),
            pl.BlockSpec((_RB, 1), lambda i: (i % (L // _RB), 0)),
            pl.BlockSpec((1, H), full),
            pl.BlockSpec((1, H), full),
            pl.BlockSpec((1, H), full),
            pl.BlockSpec((1, 3), full),
            pl.BlockSpec((H, H), full),
            pl.BlockSpec((H, H), full),
            pl.BlockSpec((1, H), full),
            pl.BlockSpec((H, H), full),
            pl.BlockSpec((1, H), full),
            pl.BlockSpec((1, H), full),
            pl.BlockSpec((1, H), full),
            pl.BlockSpec((1, H), full),
            pl.BlockSpec((1, 1), full),
            pl.BlockSpec((1, 1), full),
        ],
        out_specs=pl.BlockSpec((_RB, 1), row),
        out_shape=jax.ShapeDtypeStruct((B * L, 1), jnp.float32),
    )(x2d, agg, deg2d, wcg, bcg, ln1_b, s3,
      upd_wh, upd_wa, upd_b, fc2_w, fc2_b, ln2_g, ln2_b,
      out_w_row, out_b, skip_w2d)


def kernel(logits, edge_index, fc1_w, fc1_b, ln1_g, ln1_b, msg_w, msg_b,
           upd_w, upd_b, fc2_w, fc2_b, ln2_g, ln2_b, out_w, out_b, skip_w):
    x2d = logits.reshape(B * L, 1)
    r = lambda v: v.reshape(1, H)

    w0 = fc1_w.reshape(H)
    wz = w0 - jnp.mean(w0)
    bz = fc1_b - jnp.mean(fc1_b)
    s3 = jnp.stack([jnp.mean(wz * wz), jnp.mean(wz * bz),
                    jnp.mean(bz * bz)]).reshape(1, 3)
    wcg = (wz * ln1_g).reshape(1, H)
    bcg = (bz * ln1_g).reshape(1, H)

    h_flat = _phase1(x2d, wcg, bcg, r(ln1_b), s3)

    src = edge_index[0]
    dst = edge_index[1]
    pad = E_PAD - E
    src_p = jnp.concatenate([src, jnp.zeros((pad,), jnp.int32)])
    dst_p = jnp.concatenate([dst, jnp.full((pad,), L, jnp.int32)])
    src_tiles = src_p.reshape(NS, CHUNKS_PER_TILE, CHUNK)
    dst_tiles = dst_p.reshape(NS, CHUNKS_PER_TILE, CHUNK)

    zerosh = jnp.zeros((CHUNK, H), jnp.bfloat16)
    zeros16 = jnp.zeros((CHUNK, 16), jnp.float32)
    ones16 = jnp.ones((CHUNK, 16), jnp.float32)

    agg, deg16 = _phase2(h_flat, src_tiles, dst_tiles,
                         zerosh, zeros16, ones16)
    deg2d = deg16[:, :1]

    uwh = upd_w[:H].astype(jnp.bfloat16)
    uwa = (msg_w @ upd_w[H:]).astype(jnp.bfloat16)
    ub = (msg_b @ upd_w[H:] + upd_b).reshape(1, H)

    refined = _phase3(
        x2d, agg, deg2d,
        wcg, bcg, r(ln1_b), s3,
        uwh, uwa, ub,
        fc2_w.astype(jnp.bfloat16), r(fc2_b), r(ln2_g), r(ln2_b),
        out_w.reshape(1, H), out_b.reshape(1, 1), skip_w.reshape(1, 1),
    )
    return refined.reshape(B, L)


# speedup vs baseline: 1.0874x; 1.0186x over previous
import functools

import jax
import jax.numpy as jnp
from jax import lax
from jax.experimental import pallas as pl
from jax.experimental.pallas import tpu as pltpu
from jax.experimental.pallas import tpu_sc as plsc

B = 8
L = 10000
E = 160000
H = 128

NC = 2
NS = 16
CHUNK = 128
CHUNKS_PER_TILE = 80
E_PAD = NS * CHUNKS_PER_TILE * CHUNK
L_PAD = 10240
ROWS_PER_TILE = 640
B_PER_CORE = B // NC

_RB = 2000
_GRID = (B * L) // _RB

_INV_SQRT2 = 0.7071067811865476


def _gelu(x):
    return 0.5 * x * (1.0 + lax.erf(x * _INV_SQRT2))


def _node_head(x, wcg_ref, bcg_ref, bb1_ref, s3_ref):
    sww = s3_ref[0, 0]
    swb = s3_ref[0, 1]
    sbb = s3_ref[0, 2]
    r = lax.rsqrt((x * x) * sww + (2.0 * x) * swb + (sbb + 1e-5))
    h = (x * r) * wcg_ref[...] + r * bcg_ref[...] + bb1_ref[...]
    return _gelu(h)


def _phase1_body(x_ref, wcg_ref, bcg_ref, bb1_ref, s3_ref, out_ref):
    out_ref[...] = _node_head(x_ref[...], wcg_ref, bcg_ref, bb1_ref,
                              s3_ref).astype(jnp.bfloat16)


def _phase1(x2d, wcg, bcg, ln1_b, s3):
    full = lambda i: (0, 0)
    row = lambda i: (i, 0)
    return pl.pallas_call(
        _phase1_body,
        grid=(_GRID,),
        in_specs=[
            pl.BlockSpec((_RB, 1), row),
            pl.BlockSpec((1, H), full),
            pl.BlockSpec((1, H), full),
            pl.BlockSpec((1, H), full),
            pl.BlockSpec((1, 3), full),
        ],
        out_specs=pl.BlockSpec((_RB, H), row),
        out_shape=jax.ShapeDtypeStruct((B * L, H), jnp.bfloat16),
    )(x2d, wcg, bcg, ln1_b, s3)


HH = H // 2


NSLOT = 4


def _make_sc_body(call_base, with_deg, bpc):
    def body(msg_hbm, src_hbm, dst_hbm, zerosh_hbm, zeros16_hbm, ones16_hbm,
             *rest):
        if with_deg:
            agg_hbm, deg_hbm = rest[0], rest[1]
            scr = rest[2:]
        else:
            agg_hbm = rest[0]
            deg_hbm = None
            scr = rest[1:]
        (src_v, sabs_v, dst_v, gbuf_v, zerosh_v, zeros16_v, ones16_v,
         gsem_a, ssem_a, agg_sh, deg_sh) = scr
        gsems = [gsem_a.at[t] for t in range(NSLOT)]
        ssems = [ssem_a.at[t] for t in range(NSLOT)]
        c = lax.axis_index("c")
        s = lax.axis_index("s")
        row0 = s * ROWS_PER_TILE
        n_last = L - (NS - 1) * ROWS_PER_TILE

        pltpu.sync_copy(src_hbm.at[s], src_v)
        pltpu.sync_copy(dst_hbm.at[s], dst_v)
        pltpu.sync_copy(zerosh_hbm, zerosh_v)
        pltpu.sync_copy(zeros16_hbm, zeros16_v)
        pltpu.sync_copy(ones16_hbm, ones16_v)

        if with_deg:
            @pl.when(c == 0)
            def _deg():
                for j in range(5):
                    pltpu.sync_copy(zeros16_v,
                                    deg_sh.at[pl.ds(row0 + j * CHUNK, CHUNK)])
                plsc.subcore_barrier()

                def deg_chunk(i, carry):
                    pltpu.sync_copy(ones16_v, deg_sh.at[dst_v.at[i]], add=True)
                    return carry
                lax.fori_loop(0, CHUNKS_PER_TILE, deg_chunk, 0)
                plsc.subcore_barrier()

                @pl.when(s < NS - 1)
                def _():
                    pltpu.sync_copy(deg_sh.at[pl.ds(row0, ROWS_PER_TILE)],
                                    deg_hbm.at[pl.ds(row0, ROWS_PER_TILE)])

                @pl.when(s == NS - 1)
                def _():
                    pltpu.sync_copy(deg_sh.at[pl.ds(row0, n_last)],
                                    deg_hbm.at[pl.ds(row0, n_last)])

        for bb in range(bpc):
            b = c * B_PER_CORE + call_base + bb
            bo = c * bpc + bb

            for j in range(5):
                pltpu.sync_copy(zerosh_v,
                                agg_sh.at[pl.ds(row0 + j * CHUNK, CHUNK)])

            boff = (b * L).astype(jnp.int32)

            def absrow(i, carry):
                for k in range(CHUNK // 16):
                    sabs_v[i, pl.ds(k * 16, 16)] = (
                        src_v[i, pl.ds(k * 16, 16)] + boff)
                return carry
            lax.fori_loop(0, CHUNKS_PER_TILE, absrow, 0)
            plsc.subcore_barrier()

            def _slot(t):
                return gbuf_v.at[pl.ds(t * CHUNK, CHUNK)]

            def _gissue(j, t):
                pltpu.async_copy(msg_hbm.at[sabs_v.at[j]], _slot(t), gsems[t])

            def _gwait(j, t):
                pltpu.make_async_copy(msg_hbm.at[sabs_v.at[j]], _slot(t),
                                      gsems[t]).wait()

            def _sissue(j, t):
                pltpu.async_copy(_slot(t), agg_sh.at[dst_v.at[j]], ssems[t],
                                 add=True)

            def _swait(j, t):
                pltpu.make_async_copy(_slot(t), agg_sh.at[dst_v.at[j]],
                                      ssems[t]).wait()

            for t in range(NSLOT):
                _gissue(t, t)

            def edge_step(i, carry):
                base = i * NSLOT
                for t in range(NSLOT):
                    _gwait(base + t, t)
                    _sissue(base + t, t)

                @pl.when(i < CHUNKS_PER_TILE // NSLOT - 1)
                def _():
                    for t in range(NSLOT):
                        _swait(base + t, t)
                        _gissue(base + NSLOT + t, t)
                return carry
            lax.fori_loop(0, CHUNKS_PER_TILE // NSLOT, edge_step, 0)
            for t in range(NSLOT):
                _swait(CHUNKS_PER_TILE - NSLOT + t, t)
            plsc.subcore_barrier()

            hbase = bo * L + row0

            @pl.when(s < NS - 1)
            def _():
                pltpu.sync_copy(agg_sh.at[pl.ds(row0, ROWS_PER_TILE)],
                                agg_hbm.at[pl.ds(hbase, ROWS_PER_TILE)])

            @pl.when(s == NS - 1)
            def _():
                pltpu.sync_copy(agg_sh.at[pl.ds(row0, n_last)],
                                agg_hbm.at[pl.ds(hbase, n_last)])

            plsc.subcore_barrier()
    return body


def _phase2(msgbf, src_tiles, dst_tiles, zerosh, zeros16, ones16,
            call_base, with_deg, bpc):
    mesh = plsc.VectorSubcoreMesh(core_axis_name="c", subcore_axis_name="s")
    out_type = [jax.ShapeDtypeStruct((NC * bpc * L, H), jnp.bfloat16)]
    if with_deg:
        out_type.append(jax.ShapeDtypeStruct((L, 16), jnp.float32))
    f = pl.kernel(
        _make_sc_body(call_base, with_deg, bpc),
        out_type=out_type,
        mesh=mesh,
        scratch_types=[
            pltpu.VMEM((CHUNKS_PER_TILE, CHUNK), jnp.int32),
            pltpu.VMEM((CHUNKS_PER_TILE, CHUNK), jnp.int32),
            pltpu.VMEM((CHUNKS_PER_TILE, CHUNK), jnp.int32),
            pltpu.VMEM((NSLOT * CHUNK, H), jnp.bfloat16),
            pltpu.VMEM((CHUNK, H), jnp.bfloat16),
            pltpu.VMEM((CHUNK, 16), jnp.float32),
            pltpu.VMEM((CHUNK, 16), jnp.float32),
            pltpu.SemaphoreType.DMA((NSLOT,)),
            pltpu.SemaphoreType.DMA((NSLOT,)),
            pltpu.VMEM_SHARED((L_PAD, H), jnp.bfloat16),
            pltpu.VMEM_SHARED((L_PAD, 16), jnp.float32),
        ],
        compiler_params=pltpu.CompilerParams(use_tc_tiling_on_sc=False),
    )
    return f(msgbf, src_tiles, dst_tiles, zerosh, zeros16, ones16)


def _phase3_body(x_ref, agg_ref, deg_ref,
                 wcg_ref, bcg_ref, bb1_ref, s3_ref,
                 uwh_ref, uwa_ref, ub_ref,
                 f2w_ref, f2b_ref, g2_ref, bb2_ref,
                 ow_ref, ob_ref, sk_ref, out_ref):
    x = x_ref[...]
    h = _node_head(x, wcg_ref, bcg_ref, bb1_ref, s3_ref)

    invdeg = 1.0 / jnp.maximum(deg_ref[...], 1.0)
    m = jnp.dot(agg_ref[...], uwa_ref[...], preferred_element_type=jnp.float32)
    u = (jnp.dot(h.astype(jnp.bfloat16), uwh_ref[...],
                 preferred_element_type=jnp.float32)
         + m * invdeg + ub_ref[...])
    u = _gelu(u)

    h2 = jnp.dot(u.astype(jnp.bfloat16), f2w_ref[...],
                 preferred_element_type=jnp.float32) + f2b_ref[...]
    mu2 = jnp.mean(h2, axis=-1, keepdims=True)
    var2 = jnp.mean((h2 - mu2) ** 2, axis=-1, keepdims=True)
    h2 = (h2 - mu2) * lax.rsqrt(var2 + 1e-5) * g2_ref[...] + bb2_ref[...]
    h2 = h2 + u
    h2 = _gelu(h2)

    refined = jnp.sum(h2 * ow_ref[...], axis=-1, keepdims=True) + ob_ref[...]
    alpha = jax.nn.sigmoid(sk_ref[...])
    out_ref[...] = alpha * refined + (1.0 - alpha) * x


def _phase3(x2d, agg, deg2d, wcg, bcg, ln1_b, s3,
            upd_wh, upd_wa, upd_b, fc2_w, fc2_b, ln2_g, ln2_b,
            out_w_row, out_b, skip_w2d):
    rows = x2d.shape[0]
    full = lambda i: (0, 0)
    row = lambda i: (i, 0)
    return pl.pallas_call(
        _phase3_body,
        grid=(rows // _RB,),
        in_specs=[
            pl.BlockSpec((_RB, 1), row),
            pl.BlockSpec((_RB, H), row),
            pl.BlockSpec((_RB, 1), lambda i: (i % (L // _RB), 0)),
            pl.BlockSpec((1, H), full),
            pl.BlockSpec((1, H), full),
            pl.BlockSpec((1, H), full),
            pl.BlockSpec((1, 3), full),
            pl.BlockSpec((H, H), full),
            pl.BlockSpec((H, H), full),
            pl.BlockSpec((1, H), full),
            pl.BlockSpec((H, H), full),
            pl.BlockSpec((1, H), full),
            pl.BlockSpec((1, H), full),
            pl.BlockSpec((1, H), full),
            pl.BlockSpec((1, H), full),
            pl.BlockSpec((1, 1), full),
            pl.BlockSpec((1, 1), full),
        ],
        out_specs=pl.BlockSpec((_RB, 1), row),
        out_shape=jax.ShapeDtypeStruct((rows, 1), jnp.float32),
    )(x2d, agg, deg2d, wcg, bcg, ln1_b, s3,
      upd_wh, upd_wa, upd_b, fc2_w, fc2_b, ln2_g, ln2_b,
      out_w_row, out_b, skip_w2d)


def kernel(logits, edge_index, fc1_w, fc1_b, ln1_g, ln1_b, msg_w, msg_b,
           upd_w, upd_b, fc2_w, fc2_b, ln2_g, ln2_b, out_w, out_b, skip_w):
    x2d = logits.reshape(B * L, 1)
    r = lambda v: v.reshape(1, H)

    w0 = fc1_w.reshape(H)
    wz = w0 - jnp.mean(w0)
    bz = fc1_b - jnp.mean(fc1_b)
    s3 = jnp.stack([jnp.mean(wz * wz), jnp.mean(wz * bz),
                    jnp.mean(bz * bz)]).reshape(1, 3)
    wcg = (wz * ln1_g).reshape(1, H)
    bcg = (bz * ln1_g).reshape(1, H)

    h_flat = _phase1(x2d, wcg, bcg, r(ln1_b), s3)

    src = edge_index[0]
    dst = edge_index[1]
    pad = E_PAD - E
    src_p = jnp.concatenate([src, jnp.zeros((pad,), jnp.int32)])
    dst_p = jnp.concatenate([dst, jnp.full((pad,), L, jnp.int32)])
    src_tiles = src_p.reshape(NS, CHUNKS_PER_TILE, CHUNK)
    dst_tiles = dst_p.reshape(NS, CHUNKS_PER_TILE, CHUNK)

    zerosh = jnp.zeros((CHUNK, H), jnp.bfloat16)
    zeros16 = jnp.zeros((CHUNK, 16), jnp.float32)
    ones16 = jnp.ones((CHUNK, 16), jnp.float32)

    aggA, deg16 = _phase2(h_flat, src_tiles, dst_tiles,
                          zerosh, zeros16, ones16,
                          call_base=0, with_deg=True, bpc=2)
    (aggB,) = _phase2(h_flat, src_tiles, dst_tiles,
                      zerosh, zeros16, ones16,
                      call_base=2, with_deg=False, bpc=2)
    deg2d = deg16[:, :1]

    uwh = upd_w[:H].astype(jnp.bfloat16)
    uwa = (msg_w @ upd_w[H:]).astype(jnp.bfloat16)
    ub = (msg_b @ upd_w[H:] + upd_b).reshape(1, H)

    xA = jnp.concatenate([x2d[:2 * L], x2d[4 * L:6 * L]], axis=0)
    xB = jnp.concatenate([x2d[2 * L:4 * L], x2d[6 * L:]], axis=0)

    common = (wcg, bcg, r(ln1_b), s3, uwh, uwa, ub,
              fc2_w.astype(jnp.bfloat16), r(fc2_b), r(ln2_g), r(ln2_b),
              out_w.reshape(1, H), out_b.reshape(1, 1), skip_w.reshape(1, 1))
    refA = _phase3(xA, aggA, deg2d, *common)
    refB = _phase3(xB, aggB, deg2d, *common)

    refined = jnp.concatenate(
        [refA[:2 * L], refB[:2 * L], refA[2 * L:], refB[2 * L:]], axis=0)
    return refined.reshape(B, L)
